# baseline (speedup 1.0000x reference)
import functools

import jax
import jax.numpy as jnp
from jax import lax
from jax.experimental import pallas as pl
from jax.experimental.pallas import tpu as pltpu
from jax.experimental.pallas import tpu_sc as plsc

_B = 128
_FDIM = 32768
_K = 2048
_L = 16
_NV = _FDIM // _L
_KV = _K // _L
_CAP = 4096
_CV = _CAP // _L
_MIN32 = -2147483648
_MAX32 = 2147483647
_UNROLL = 4


def _kd_from_f32(f):
    b = plsc.bitcast(f, jnp.int32)
    ki = b ^ (lax.shift_right_arithmetic(b, 31) & 0x7FFFFFFF)
    return jnp.bitwise_not(ki ^ _MIN32)


def _f32_from_kd(kd):
    ki = jnp.bitwise_not(kd) ^ _MIN32
    b = ki ^ (lax.shift_right_arithmetic(ki, 31) & 0x7FFFFFFF)
    return plsc.bitcast(b, jnp.float32)


def _topk_body(x_hbm, out_hbm, xv, hist, h2, bufa, bufb, outv, smem, sem):
    del sem
    cid = lax.axis_index("c")
    sid = lax.axis_index("s")
    wid = sid * 2 + cid
    row0 = wid * (_B // 32)

    iota = lax.iota(jnp.int32, _L)
    true16 = iota < 16
    ones16 = iota * 0 + 1
    zeros16 = iota * 0

    def zero_h2():
        def zbody(i, _):
            for u in range(_UNROLL):
                h2[pl.ds((i * _UNROLL + u) * _L, _L)] = zeros16
            return 0
        lax.fori_loop(0, 1024 // _UNROLL, zbody, 0)

    def zero_hist(nbins):
        zv = jnp.zeros((_L,), jnp.int32)
        def zbody(i, _):
            for u in range(_UNROLL):
                hist[pl.ds((i * _UNROLL + u) * _L, _L)] = zv
            return 0
        lax.fori_loop(0, nbins // _L // _UNROLL, zbody, 0)

    def kd_at(i):
        return _kd_from_f32(xv[pl.ds(i * _L, _L)])

    def hist_sweep(shift, maskbits, pshift, pval):
        def body(i, _):
            for u in range(_UNROLL):
                kd = kd_at(i * _UNROLL + u)
                d = lax.shift_right_logical(kd, shift) & maskbits
                if pshift is None:
                    c, lm = plsc.scan_count(d)
                    plsc.addupdate_scatter(hist, [d], c, mask=lm)
                else:
                    elig = lax.shift_right_logical(kd, pshift) == pval
                    c, lm = plsc.scan_count(d, mask=elig)
                    plsc.addupdate_scatter(hist, [d], c, mask=lm & elig)
            return 0
        lax.fori_loop(0, _NV // _UNROLL, body, 0)

    def search(nbins, need):
        def body(i, carry):
            tot, bstar, cumex = carry
            h = hist[pl.ds(i * _L, _L)]
            cin = lax.cumsum(h, axis=0) + tot
            m = cin >= need
            idxv = iota + i * _L
            bstar = jnp.minimum(bstar, jnp.min(jnp.where(m, idxv, _MAX32)))
            cumex = jnp.minimum(cumex, jnp.min(jnp.where(m, cin - h, _MAX32)))
            tot = jnp.max(cin)
            return tot, bstar, cumex
        init = (jnp.int32(0), jnp.int32(_MAX32), jnp.int32(_MAX32))
        _, bstar, cumex = lax.fori_loop(0, nbins // _L, body, init)
        hb = jnp.max(plsc.load_gather(hist, [jnp.zeros((_L,), jnp.int32) + bstar]))
        return bstar, cumex, hb

    def row_body(r, _):
        row = row0 + r
        pltpu.sync_copy(x_hbm.at[row], xv)

        scope = jax.named_scope("ph_hist1"); scope.__enter__()
        def h1body(i, _):
            for u in range(_UNROLL):
                kd = kd_at(i * _UNROLL + u)
                d = lax.shift_right_logical(kd, 22)
                plsc.addupdate_scatter(h2, [lax.shift_left(d, 4) | iota],
                                       ones16)
            return 0
        lax.fori_loop(0, _NV // _UNROLL, h1body, 0)
        scope.__exit__(None, None, None)
        scope = jax.named_scope("ph_search1"); scope.__enter__()
        def g1body(j, carry):
            base, cum, jstar, cumbase = carry
            vs = h2[pl.ds(base, _L)]
            for t in range(1, 16):
                vs = vs + h2[pl.ds(base + t * _L, _L)]
            cum2 = cum + jnp.max(lax.cumsum(vs, axis=0))
            hit = cum2 >= _K
            jstar = jnp.minimum(jstar, jnp.where(hit, j, _MAX32))
            cumbase = jnp.minimum(cumbase, jnp.where(hit, cum, _MAX32))
            return base + 256, cum2, jstar, cumbase
        _, _, jstar, cumbase = lax.fori_loop(
            0, 64, g1body,
            (jnp.int32(0), jnp.int32(0), jnp.int32(_MAX32),
             jnp.int32(_MAX32)))
        gbase = jstar * 256
        cumr = cumbase
        b1 = jnp.int32(_MAX32)
        c1 = jnp.int32(_MAX32)
        cuminc = jnp.int32(_MAX32)
        for t in range(16):
            sk = jnp.max(lax.cumsum(h2[pl.ds(gbase + t * _L, _L)], axis=0))
            cumn = cumr + sk
            hit = cumn >= _K
            b1 = jnp.minimum(b1, jnp.where(hit, jstar * 16 + t, _MAX32))
            c1 = jnp.minimum(c1, jnp.where(hit, cumr, _MAX32))
            cuminc = jnp.minimum(cuminc, jnp.where(hit, cumn, _MAX32))
            cumr = cumn
        s1 = cuminc - c1
        zero_h2()
        scope.__exit__(None, None, None)
        u1 = lax.shift_left(b1 + 1, 22) - 1
        smem[0] = u1
        smem[1] = u1
        @pl.when(c1 + s1 > _CAP)
        def _l2():
            zero_hist(2048)
            hist_sweep(11, 0x7FF, 22, b1)
            b2, c2x, s2 = search(2048, _K - c1)
            p2 = lax.shift_left(b1, 11) | b2
            u2 = lax.shift_left(p2 + 1, 11) - 1
            smem[0] = u2
            smem[1] = u2
            @pl.when(c1 + c2x + s2 > _CAP)
            def _l3():
                zero_hist(2048)
                hist_sweep(0, 0x7FF, 11, p2)
                b3, _, _ = search(2048, _K - (c1 + c2x))
                thr = lax.shift_left(p2, 11) | b3
                smem[0] = thr - 1
                smem[1] = thr

        ubound = smem[0]
        pad = smem[1]

        us = ubound ^ _MIN32
        pad_s = pad ^ _MIN32
        scope = jax.named_scope("ph_compact"); scope.__enter__()
        padv_s = jnp.zeros((_L,), jnp.int32) + pad_s
        def cbody(i, carry):
            off, kmin_s = carry
            acc = zeros16
            kv = padv_s
            for u in range(_UNROLL):
                kd = kd_at(i * _UNROLL + u)
                kds = kd ^ _MIN32
                m = kds <= us
                pos = off + (acc + plsc.cumsum(m.astype(jnp.int32)) - 1)
                plsc.store_scatter(bufa, [pos], kd, mask=m)
                acc = acc + plsc.all_reduce_population_count(m)
                kv = jnp.minimum(kv, jnp.where(m, kds, padv_s))
            off = off + jnp.max(acc)
            kmin_s = jnp.minimum(kmin_s, jnp.min(kv))
            return off, kmin_s
        off, kmin_s = lax.fori_loop(0, _NV // _UNROLL, cbody,
                                    (jnp.int32(0), pad_s))
        kmin = kmin_s ^ _MIN32

        nv_sort = jnp.maximum(
            (lax.shift_right_logical(off + (_L - 1), 4) + 3) & ~3,
            jnp.int32(_KV))
        padvec = jnp.zeros((_L,), jnp.int32) + pad
        plsc.store_compressed(bufa.at[pl.ds(off, _L)], padvec, mask=true16)
        def fbody(j, _):
            plsc.store_compressed(bufa.at[pl.ds(j * _L, _L)], padvec,
                                  mask=true16)
            return 0
        lax.fori_loop(lax.shift_right_logical(off, 4) + 1, nv_sort, fbody, 0)
        scope.__exit__(None, None, None)

        rng = pad - kmin

        def do_pass(src, dst, p):
            zero_hist(256)
            def hbody(i, _):
                for u in range(_UNROLL):
                    v = src[pl.ds((i * _UNROLL + u) * _L, _L)]
                    if p == 0:
                        v = v - kmin
                    d = lax.shift_right_logical(v, 8 * p) & 0xFF
                    c, lm = plsc.scan_count(d)
                    plsc.addupdate_scatter(hist, [d], c, mask=lm)
                return 0
            lax.fori_loop(0, nv_sort // _UNROLL, hbody, 0)
            def pfx(i, tot):
                h = hist[pl.ds(i * _L, _L)]
                cum = lax.cumsum(h, axis=0) + tot
                hist[pl.ds(i * _L, _L)] = cum - h
                return jnp.max(cum)
            lax.fori_loop(0, 256 // _L, pfx, jnp.int32(0))
            def sbody(i, _):
                for u in range(2):
                    v = src[pl.ds((i * 2 + u) * _L, _L)]
                    if p == 0:
                        v = v - kmin
                    d = lax.shift_right_logical(v, 8 * p) & 0xFF
                    c, lm = plsc.scan_count(d)
                    base = plsc.load_gather(hist, [d])
                    plsc.store_scatter(dst, [base + c - 1], v)
                    plsc.addupdate_scatter(hist, [d], c, mask=lm)
                return 0
            lax.fori_loop(0, nv_sort // 2, sbody, 0)

        with jax.named_scope("ph_sort0"):
            do_pass(bufa, bufb, 0)
        with jax.named_scope("ph_sort1"):
            @pl.when(lax.shift_right_logical(rng, 8) != 0)
            def _p1():
                do_pass(bufb, bufa, 1)
        with jax.named_scope("ph_sort2"):
            @pl.when(lax.shift_right_logical(rng, 16) != 0)
            def _p2():
                do_pass(bufa, bufb, 2)
        @pl.when(lax.shift_right_logical(rng, 24) != 0)
        def _p3():
            do_pass(bufb, bufa, 3)

        npasses = (jnp.int32(1)
                   + (lax.shift_right_logical(rng, 8) != 0).astype(jnp.int32)
                   + (lax.shift_right_logical(rng, 16) != 0).astype(jnp.int32)
                   + (lax.shift_right_logical(rng, 24) != 0).astype(jnp.int32))

        def emit_from(src):
            def ebody(i, _):
                for u in range(_UNROLL):
                    j = i * _UNROLL + u
                    outv[pl.ds(j * _L, _L)] = _f32_from_kd(
                        src[pl.ds(j * _L, _L)] + kmin)
                return 0
            lax.fori_loop(0, _KV // _UNROLL, ebody, 0)

        scope = jax.named_scope("ph_emit"); scope.__enter__()
        @pl.when(npasses % 2 == 1)
        def _ea():
            emit_from(bufb)
        @pl.when(npasses % 2 == 0)
        def _eb():
            emit_from(bufa)
        pltpu.sync_copy(outv, out_hbm.at[row])
        scope.__exit__(None, None, None)
        return 0

    zero_h2()
    lax.fori_loop(0, _B // 32, row_body, 0)


def _topk_vals(x):
    mesh = plsc.VectorSubcoreMesh(core_axis_name="c", subcore_axis_name="s")
    call = functools.partial(
        pl.kernel,
        out_type=jax.ShapeDtypeStruct((_B, _K), jnp.float32),
        mesh=mesh,
        scratch_types=[
            pltpu.VMEM((_FDIM,), jnp.float32),
            pltpu.VMEM((2048,), jnp.int32),
            pltpu.VMEM((1024 * _L,), jnp.int32),
            pltpu.VMEM((_CAP + 2 * _L,), jnp.int32),
            pltpu.VMEM((_CAP + 2 * _L,), jnp.int32),
            pltpu.VMEM((_K,), jnp.float32),
            pltpu.SMEM((4,), jnp.int32),
            pltpu.SemaphoreType.DMA,
        ],
        compiler_params=pltpu.CompilerParams(needs_layout_passes=False),
    )(_topk_body)
    return call(x)


def kernel(x, cls_token):
    vals = _topk_vals(x)
    rank = jnp.broadcast_to(
        jnp.linspace(0.0, 1.0, _K, dtype=jnp.float32)[None, :], (_B, _K))
    dropout = (vals == 0).astype(jnp.float32)
    tokens = jnp.stack([vals, rank, dropout], axis=-1)
    cls = jnp.broadcast_to(cls_token, (_B, 1, tokens.shape[-1]))
    return jnp.concatenate([cls, tokens], axis=1)


# speedup vs baseline: 2.1638x; 2.1638x over previous
import functools

import jax
import jax.numpy as jnp
from jax import lax
from jax.experimental import pallas as pl
from jax.experimental.pallas import tpu as pltpu
from jax.experimental.pallas import tpu_sc as plsc

_B = 128
_FDIM = 32768
_K = 2048
_L = 16
_NV = _FDIM // _L
_KV = _K // _L
_CAP = 4096
_MIN32 = -2147483648
_MAX32 = 2147483647
_UNROLL = 4


def _topk_body(x_hbm, out_hbm, xv, hist, h2, bufa, bufb, outv, smem, sem):
    del sem
    cid = lax.axis_index("c")
    sid = lax.axis_index("s")
    wid = sid * 2 + cid
    row0 = wid * (_B // 32)

    iota = lax.iota(jnp.int32, _L)
    ones16 = iota * 0 + 1
    zeros16 = iota * 0

    def kd_batch(refv, i0, n):
        fs = [refv[pl.ds((i0 + u) * _L, _L)] for u in range(n)]
        bs = [plsc.bitcast(f, jnp.int32) for f in fs]
        sg = [lax.shift_right_arithmetic(b, 31) for b in bs]
        sg = [s & 0x7FFFFFFF for s in sg]
        ki = [b ^ s for b, s in zip(bs, sg)]
        return [jnp.bitwise_not(k ^ _MIN32) for k in ki]

    def zero_h2():
        def zbody(i, _):
            for u in range(_UNROLL):
                h2[pl.ds((i * _UNROLL + u) * _L, _L)] = zeros16
            return 0
        lax.fori_loop(0, 1024 // _UNROLL, zbody, 0)

    def zero_hist(nbins):
        def zbody(i, _):
            for u in range(_UNROLL):
                hist[pl.ds((i * _UNROLL + u) * _L, _L)] = zeros16
            return 0
        lax.fori_loop(0, nbins // _L // _UNROLL, zbody, 0)

    def hist_sweep(shift, maskbits, pshift, pval):
        def body(i, _):
            kds = kd_batch(xv, i * _UNROLL, _UNROLL)
            ds = [lax.shift_right_logical(kd, shift) & maskbits for kd in kds]
            els = [lax.shift_right_logical(kd, pshift) == pval for kd in kds]
            scs = [plsc.scan_count(d, mask=e) for d, e in zip(ds, els)]
            for (c, lm), d, e in zip(scs, ds, els):
                plsc.addupdate_scatter(hist, [d], c, mask=lm & e)
            return 0
        lax.fori_loop(0, _NV // _UNROLL, body, 0)

    def search(nbins, need):
        def body(i, carry):
            tot, bstar, cumex = carry
            h = hist[pl.ds(i * _L, _L)]
            cin = lax.cumsum(h, axis=0) + tot
            m = cin >= need
            idxv = iota + i * _L
            bstar = jnp.minimum(bstar, jnp.min(jnp.where(m, idxv, _MAX32)))
            cumex = jnp.minimum(cumex, jnp.min(jnp.where(m, cin - h, _MAX32)))
            tot = jnp.max(cin)
            return tot, bstar, cumex
        init = (jnp.int32(0), jnp.int32(_MAX32), jnp.int32(_MAX32))
        _, bstar, cumex = lax.fori_loop(0, nbins // _L, body, init)
        hb = jnp.max(plsc.load_gather(
            hist, [jnp.zeros((_L,), jnp.int32) + bstar]))
        return bstar, cumex, hb

    def row_body(r, _):
        row = row0 + r
        pltpu.sync_copy(x_hbm.at[row], xv)

        scope = jax.named_scope("ph_hist1"); scope.__enter__()
        def h1body(i, _):
            kds = kd_batch(xv, i * _UNROLL, _UNROLL)
            ds = [lax.shift_right_logical(kd, 22) for kd in kds]
            ix = [lax.shift_left(d, 4) | iota for d in ds]
            for x in ix:
                plsc.addupdate_scatter(h2, [x], ones16)
            return 0
        lax.fori_loop(0, _NV // _UNROLL, h1body, 0)
        scope.__exit__(None, None, None)

        scope = jax.named_scope("ph_search1"); scope.__enter__()
        def g1body(j, carry):
            base, cum, jstar, cumbase = carry
            vs = [h2[pl.ds(base + t * _L, _L)] for t in range(16)]
            while len(vs) > 1:
                vs = [a + b for a, b in zip(vs[::2], vs[1::2])]
            cum2 = cum + jnp.max(lax.cumsum(vs[0], axis=0))
            hit = cum2 >= _K
            jstar = jnp.minimum(jstar, jnp.where(hit, j, _MAX32))
            cumbase = jnp.minimum(cumbase, jnp.where(hit, cum, _MAX32))
            return base + 256, cum2, jstar, cumbase
        _, _, jstar, cumbase = lax.fori_loop(
            0, 64, g1body,
            (jnp.int32(0), jnp.int32(0), jnp.int32(_MAX32),
             jnp.int32(_MAX32)))
        gbase = jstar * 256
        cumr = cumbase
        b1 = jnp.int32(_MAX32)
        c1 = jnp.int32(_MAX32)
        cuminc = jnp.int32(_MAX32)
        for t in range(16):
            sk = jnp.max(lax.cumsum(h2[pl.ds(gbase + t * _L, _L)], axis=0))
            cumn = cumr + sk
            hit = cumn >= _K
            b1 = jnp.minimum(b1, jnp.where(hit, jstar * 16 + t, _MAX32))
            c1 = jnp.minimum(c1, jnp.where(hit, cumr, _MAX32))
            cuminc = jnp.minimum(cuminc, jnp.where(hit, cumn, _MAX32))
            cumr = cumn
        s1 = cuminc - c1
        zero_h2()
        scope.__exit__(None, None, None)

        u1 = lax.shift_left(b1 + 1, 22) - 1
        smem[0] = u1
        smem[1] = u1
        @pl.when(c1 + s1 > _CAP)
        def _l2():
            zero_hist(2048)
            hist_sweep(11, 0x7FF, 22, b1)
            b2, c2x, s2 = search(2048, _K - c1)
            p2 = lax.shift_left(b1, 11) | b2
            u2 = lax.shift_left(p2 + 1, 11) - 1
            smem[0] = u2
            smem[1] = u2
            @pl.when(c1 + c2x + s2 > _CAP)
            def _l3():
                zero_hist(2048)
                hist_sweep(0, 0x7FF, 11, p2)
                b3, _, _ = search(2048, _K - (c1 + c2x))
                thr = lax.shift_left(p2, 11) | b3
                smem[0] = thr - 1
                smem[1] = thr

        ubound = smem[0]
        pad = smem[1]

        us = ubound ^ _MIN32
        pad_s = pad ^ _MIN32
        scope = jax.named_scope("ph_compact"); scope.__enter__()
        padv_s = jnp.zeros((_L,), jnp.int32) + pad_s
        def cbody(i, carry):
            off, kmin_s = carry
            kds = kd_batch(xv, i * _UNROLL, _UNROLL)
            kss = [kd ^ _MIN32 for kd in kds]
            ms = [ks <= us for ks in kss]
            cums = [plsc.cumsum(m.astype(jnp.int32)) for m in ms]
            pcs = [plsc.all_reduce_population_count(m) for m in ms]
            kvs = [jnp.where(m, ks, padv_s) for m, ks in zip(ms, kss)]
            acc = zeros16
            poss = []
            for u in range(_UNROLL):
                poss.append(off + (acc + (cums[u] - 1)))
                acc = acc + pcs[u]
            for u in range(_UNROLL):
                plsc.store_scatter(bufa, [poss[u]], kds[u], mask=ms[u])
            kv = jnp.minimum(jnp.minimum(kvs[0], kvs[1]),
                             jnp.minimum(kvs[2], kvs[3]))
            off = off + jnp.max(acc)
            kmin_s = jnp.minimum(kmin_s, jnp.min(kv))
            return off, kmin_s
        off, kmin_s = lax.fori_loop(0, _NV // _UNROLL, cbody,
                                    (jnp.int32(0), pad_s))
        kmin = kmin_s ^ _MIN32

        nv_sort = jnp.maximum(
            (lax.shift_right_logical(off + (_L - 1), 4) + 3) & ~3,
            jnp.int32(_KV))
        padvec = jnp.zeros((_L,), jnp.int32) + pad
        plsc.store_scatter(bufa, [off + iota], padvec)
        def fbody(j, _):
            bufa[pl.ds(j * _L, _L)] = padvec
            return 0
        lax.fori_loop(lax.shift_right_logical(off, 4) + 1, nv_sort, fbody, 0)
        scope.__exit__(None, None, None)

        rng = pad - kmin

        def do_pass(src, dst, p):
            zero_hist(256)
            def hbody(i, _):
                vs = [src[pl.ds((i * _UNROLL + u) * _L, _L)]
                      for u in range(_UNROLL)]
                if p == 0:
                    vs = [v - kmin for v in vs]
                ds = [lax.shift_right_logical(v, 8 * p) & 0xFF for v in vs]
                scs = [plsc.scan_count(d) for d in ds]
                for (c, lm), d in zip(scs, ds):
                    plsc.addupdate_scatter(hist, [d], c, mask=lm)
                return 0
            lax.fori_loop(0, nv_sort // _UNROLL, hbody, 0)
            def pfx(i, tot):
                h = hist[pl.ds(i * _L, _L)]
                cum = lax.cumsum(h, axis=0) + tot
                hist[pl.ds(i * _L, _L)] = cum - h
                return jnp.max(cum)
            lax.fori_loop(0, 256 // _L, pfx, jnp.int32(0))
            def sbody(i, _):
                vs = [src[pl.ds((i * 2 + u) * _L, _L)] for u in range(2)]
                if p == 0:
                    vs = [v - kmin for v in vs]
                ds = [lax.shift_right_logical(v, 8 * p) & 0xFF for v in vs]
                scs = [plsc.scan_count(d) for d in ds]
                for (c, lm), d, v in zip(scs, ds, vs):
                    base = plsc.load_gather(hist, [d])
                    plsc.store_scatter(dst, [base + c - 1], v)
                    plsc.addupdate_scatter(hist, [d], c, mask=lm)
                return 0
            lax.fori_loop(0, nv_sort // 2, sbody, 0)

        with jax.named_scope("ph_sort0"):
            do_pass(bufa, bufb, 0)
        with jax.named_scope("ph_sort1"):
            @pl.when(lax.shift_right_logical(rng, 8) != 0)
            def _p1():
                do_pass(bufb, bufa, 1)
        with jax.named_scope("ph_sort2"):
            @pl.when(lax.shift_right_logical(rng, 16) != 0)
            def _p2():
                do_pass(bufa, bufb, 2)
        with jax.named_scope("ph_sort3"):
            @pl.when(lax.shift_right_logical(rng, 24) != 0)
            def _p3():
                do_pass(bufb, bufa, 3)

        npasses = (jnp.int32(1)
                   + (lax.shift_right_logical(rng, 8) != 0).astype(jnp.int32)
                   + (lax.shift_right_logical(rng, 16) != 0).astype(jnp.int32)
                   + (lax.shift_right_logical(rng, 24) != 0).astype(jnp.int32))

        scope = jax.named_scope("ph_emit"); scope.__enter__()
        def emit_from(src):
            def ebody(i, _):
                vs = [src[pl.ds((i * _UNROLL + u) * _L, _L)]
                      for u in range(_UNROLL)]
                kd = [v + kmin for v in vs]
                ud = [jnp.bitwise_not(k) ^ _MIN32 for k in kd]
                sg = [lax.shift_right_arithmetic(k, 31) & 0x7FFFFFFF
                      for k in ud]
                bs = [k ^ s for k, s in zip(ud, sg)]
                fs = [plsc.bitcast(b, jnp.float32) for b in bs]
                for u in range(_UNROLL):
                    outv[pl.ds((i * _UNROLL + u) * _L, _L)] = fs[u]
                return 0
            lax.fori_loop(0, _KV // _UNROLL, ebody, 0)

        @pl.when(npasses % 2 == 1)
        def _ea():
            emit_from(bufb)
        @pl.when(npasses % 2 == 0)
        def _eb():
            emit_from(bufa)
        pltpu.sync_copy(outv, out_hbm.at[row])
        scope.__exit__(None, None, None)
        return 0

    zero_h2()
    lax.fori_loop(0, _B // 32, row_body, 0)


def _topk_vals(x):
    mesh = plsc.VectorSubcoreMesh(core_axis_name="c", subcore_axis_name="s")
    call = functools.partial(
        pl.kernel,
        out_type=jax.ShapeDtypeStruct((_B, _K), jnp.float32),
        mesh=mesh,
        scratch_types=[
            pltpu.VMEM((_FDIM,), jnp.float32),
            pltpu.VMEM((2048,), jnp.int32),
            pltpu.VMEM((1024 * _L,), jnp.int32),
            pltpu.VMEM((_CAP + 2 * _L,), jnp.int32),
            pltpu.VMEM((_CAP + 2 * _L,), jnp.int32),
            pltpu.VMEM((_K,), jnp.float32),
            pltpu.SMEM((4,), jnp.int32),
            pltpu.SemaphoreType.DMA,
        ],
        compiler_params=pltpu.CompilerParams(needs_layout_passes=False),
    )(_topk_body)
    return call(x)


def kernel(x, cls_token):
    vals = _topk_vals(x)
    rank = jnp.broadcast_to(
        jnp.linspace(0.0, 1.0, _K, dtype=jnp.float32)[None, :], (_B, _K))
    dropout = (vals == 0).astype(jnp.float32)
    tokens = jnp.stack([vals, rank, dropout], axis=-1)
    cls = jnp.broadcast_to(cls_token, (_B, 1, tokens.shape[-1]))
    return jnp.concatenate([cls, tokens], axis=1)


# speedup vs baseline: 2.6710x; 1.2344x over previous
import functools

import jax
import jax.numpy as jnp
from jax import lax
from jax.experimental import pallas as pl
from jax.experimental.pallas import tpu as pltpu
from jax.experimental.pallas import tpu_sc as plsc

_B = 128
_FDIM = 32768
_K = 2048
_L = 16
_NV = _FDIM // _L
_KV = _K // _L
_CAP = 4096
_MIN32 = -2147483648
_MAX32 = 2147483647
_UNROLL = 8


def _topk_body(x_hbm, out_hbm, xv, hist, h2, bufa, bufb, outv, smem, sem):
    cid = lax.axis_index("c")
    sid = lax.axis_index("s")
    wid = sid * 2 + cid
    row0 = wid * (_B // 32)

    iota = lax.iota(jnp.int32, _L)
    ones16 = iota * 0 + 1
    zeros16 = iota * 0

    def kd_batch(refv, i0, n):
        fs = [refv[pl.ds((i0 + u) * _L, _L)] for u in range(n)]
        bs = [plsc.bitcast(f, jnp.int32) for f in fs]
        sg = [lax.shift_right_arithmetic(b, 31) for b in bs]
        sg = [s & 0x7FFFFFFF for s in sg]
        ki = [b ^ s for b, s in zip(bs, sg)]
        return [jnp.bitwise_not(k ^ _MIN32) for k in ki]

    def zero_h2():
        def zbody(i, _):
            for u in range(_UNROLL):
                h2[pl.ds((i * _UNROLL + u) * _L, _L)] = zeros16
            return 0
        lax.fori_loop(0, 1024 // _UNROLL, zbody, 0)

    def zero_hist(nbins):
        def zbody(i, _):
            for u in range(_UNROLL):
                hist[pl.ds((i * _UNROLL + u) * _L, _L)] = zeros16
            return 0
        lax.fori_loop(0, nbins // _L // _UNROLL, zbody, 0)

    def hist_sweep(shift, maskbits, pshift, pval):
        def body(i, _):
            kds = kd_batch(xv, i * _UNROLL, _UNROLL)
            ds = [lax.shift_right_logical(kd, shift) & maskbits for kd in kds]
            els = [lax.shift_right_logical(kd, pshift) == pval for kd in kds]
            scs = [plsc.scan_count(d, mask=e) for d, e in zip(ds, els)]
            for (c, lm), d, e in zip(scs, ds, els):
                plsc.addupdate_scatter(hist, [d], c, mask=lm & e)
            return 0
        lax.fori_loop(0, _NV // _UNROLL, body, 0)

    def search(nbins, need):
        def body(i, carry):
            tot, bstar, cumex = carry
            h = hist[pl.ds(i * _L, _L)]
            cin = lax.cumsum(h, axis=0) + tot
            m = cin >= need
            idxv = iota + i * _L
            bstar = jnp.minimum(bstar, jnp.min(jnp.where(m, idxv, _MAX32)))
            cumex = jnp.minimum(cumex, jnp.min(jnp.where(m, cin - h, _MAX32)))
            tot = jnp.max(cin)
            return tot, bstar, cumex
        init = (jnp.int32(0), jnp.int32(_MAX32), jnp.int32(_MAX32))
        _, bstar, cumex = lax.fori_loop(0, nbins // _L, body, init)
        hb = jnp.max(plsc.load_gather(
            hist, [jnp.zeros((_L,), jnp.int32) + bstar]))
        return bstar, cumex, hb

    def row_body(r, _):
        row = row0 + r
        pltpu.make_async_copy(x_hbm.at[row], xv, sem).wait()

        scope = jax.named_scope("ph_hist1"); scope.__enter__()
        def h1body(i, _):
            kds = kd_batch(xv, i * _UNROLL, _UNROLL)
            ds = [lax.shift_right_logical(kd, 22) for kd in kds]
            ix = [lax.shift_left(d, 4) | iota for d in ds]
            for x in ix:
                plsc.addupdate_scatter(h2, [x], ones16)
            return 0
        lax.fori_loop(0, _NV // _UNROLL, h1body, 0)
        scope.__exit__(None, None, None)

        scope = jax.named_scope("ph_search1"); scope.__enter__()
        def g1body(j, carry):
            base, cum, jstar, cumbase = carry
            vs = [h2[pl.ds(base + t * _L, _L)] for t in range(16)]
            while len(vs) > 1:
                vs = [a + b for a, b in zip(vs[::2], vs[1::2])]
            cum2 = cum + jnp.max(lax.cumsum(vs[0], axis=0))
            hit = cum2 >= _K
            jstar = jnp.minimum(jstar, jnp.where(hit, j, _MAX32))
            cumbase = jnp.minimum(cumbase, jnp.where(hit, cum, _MAX32))
            return base + 256, cum2, jstar, cumbase
        _, _, jstar, cumbase = lax.fori_loop(
            0, 64, g1body,
            (jnp.int32(0), jnp.int32(0), jnp.int32(_MAX32),
             jnp.int32(_MAX32)))
        gbase = jstar * 256
        cumr = cumbase
        b1 = jnp.int32(_MAX32)
        c1 = jnp.int32(_MAX32)
        cuminc = jnp.int32(_MAX32)
        for t in range(16):
            sk = jnp.max(lax.cumsum(h2[pl.ds(gbase + t * _L, _L)], axis=0))
            cumn = cumr + sk
            hit = cumn >= _K
            b1 = jnp.minimum(b1, jnp.where(hit, jstar * 16 + t, _MAX32))
            c1 = jnp.minimum(c1, jnp.where(hit, cumr, _MAX32))
            cuminc = jnp.minimum(cuminc, jnp.where(hit, cumn, _MAX32))
            cumr = cumn
        s1 = cuminc - c1
        zero_h2()
        scope.__exit__(None, None, None)

        u1 = lax.shift_left(b1 + 1, 22) - 1
        smem[0] = u1
        smem[1] = u1
        @pl.when(c1 + s1 > _CAP)
        def _l2():
            zero_hist(2048)
            hist_sweep(11, 0x7FF, 22, b1)
            b2, c2x, s2 = search(2048, _K - c1)
            p2 = lax.shift_left(b1, 11) | b2
            u2 = lax.shift_left(p2 + 1, 11) - 1
            smem[0] = u2
            smem[1] = u2
            @pl.when(c1 + c2x + s2 > _CAP)
            def _l3():
                zero_hist(2048)
                hist_sweep(0, 0x7FF, 11, p2)
                b3, _, _ = search(2048, _K - (c1 + c2x))
                thr = lax.shift_left(p2, 11) | b3
                smem[0] = thr - 1
                smem[1] = thr

        ubound = smem[0]
        pad = smem[1]

        us = ubound ^ _MIN32
        pad_s = pad ^ _MIN32
        scope = jax.named_scope("ph_compact"); scope.__enter__()
        padv_s = jnp.zeros((_L,), jnp.int32) + pad_s
        def cbody(i, carry):
            off, kmin_s = carry
            kds = kd_batch(xv, i * _UNROLL, _UNROLL)
            kss = [kd ^ _MIN32 for kd in kds]
            ms = [ks <= us for ks in kss]
            cums = [plsc.cumsum(m.astype(jnp.int32)) for m in ms]
            pcs = [plsc.all_reduce_population_count(m) for m in ms]
            kvs = [jnp.where(m, ks, padv_s) for m, ks in zip(ms, kss)]
            acc = zeros16
            poss = []
            for u in range(_UNROLL):
                poss.append(off + (acc + (cums[u] - 1)))
                acc = acc + pcs[u]
            for u in range(_UNROLL):
                plsc.store_scatter(bufa, [poss[u]], kds[u], mask=ms[u])
            while len(kvs) > 1:
                kvs = [jnp.minimum(a, b) for a, b in zip(kvs[::2], kvs[1::2])]
            kv = kvs[0]
            off = off + jnp.max(acc)
            kmin_s = jnp.minimum(kmin_s, jnp.min(kv))
            return off, kmin_s
        off, kmin_s = lax.fori_loop(0, _NV // _UNROLL, cbody,
                                    (jnp.int32(0), pad_s))
        kmin = kmin_s ^ _MIN32

        nv_sort = jnp.maximum(
            (lax.shift_right_logical(off + (_L - 1), 4) + 7) & ~7,
            jnp.int32(_KV))
        padvec = jnp.zeros((_L,), jnp.int32) + pad
        plsc.store_scatter(bufa, [off + iota], padvec)
        def fbody(j, _):
            bufa[pl.ds(j * _L, _L)] = padvec
            return 0
        lax.fori_loop(lax.shift_right_logical(off, 4) + 1, nv_sort, fbody, 0)
        scope.__exit__(None, None, None)

        @pl.when(r < _B // 32 - 1)
        def _pf():
            pltpu.async_copy(x_hbm.at[row + 1], xv, sem)

        rng = pad - kmin

        def do_pass(src, dst, p):
            zero_hist(256)
            def hbody(i, _):
                vs = [src[pl.ds((i * _UNROLL + u) * _L, _L)]
                      for u in range(_UNROLL)]
                if p == 0:
                    vs = [v - kmin for v in vs]
                ds = [lax.shift_right_logical(v, 8 * p) & 0xFF for v in vs]
                scs = [plsc.scan_count(d) for d in ds]
                for (c, lm), d in zip(scs, ds):
                    plsc.addupdate_scatter(hist, [d], c, mask=lm)
                return 0
            lax.fori_loop(0, nv_sort // _UNROLL, hbody, 0)
            def pfx(i, tot):
                h = hist[pl.ds(i * _L, _L)]
                cum = lax.cumsum(h, axis=0) + tot
                hist[pl.ds(i * _L, _L)] = cum - h
                return jnp.max(cum)
            lax.fori_loop(0, 256 // _L, pfx, jnp.int32(0))
            def sbody(i, _):
                vs = [src[pl.ds((i * 2 + u) * _L, _L)] for u in range(2)]
                if p == 0:
                    vs = [v - kmin for v in vs]
                ds = [lax.shift_right_logical(v, 8 * p) & 0xFF for v in vs]
                scs = [plsc.scan_count(d) for d in ds]
                for (c, lm), d, v in zip(scs, ds, vs):
                    base = plsc.load_gather(hist, [d])
                    plsc.store_scatter(dst, [base + c - 1], v)
                    plsc.addupdate_scatter(hist, [d], c, mask=lm)
                return 0
            lax.fori_loop(0, nv_sort // 2, sbody, 0)

        with jax.named_scope("ph_sort0"):
            do_pass(bufa, bufb, 0)
        with jax.named_scope("ph_sort1"):
            @pl.when(lax.shift_right_logical(rng, 8) != 0)
            def _p1():
                do_pass(bufb, bufa, 1)
        with jax.named_scope("ph_sort2"):
            @pl.when(lax.shift_right_logical(rng, 16) != 0)
            def _p2():
                do_pass(bufa, bufb, 2)
        with jax.named_scope("ph_sort3"):
            @pl.when(lax.shift_right_logical(rng, 24) != 0)
            def _p3():
                do_pass(bufb, bufa, 3)

        npasses = (jnp.int32(1)
                   + (lax.shift_right_logical(rng, 8) != 0).astype(jnp.int32)
                   + (lax.shift_right_logical(rng, 16) != 0).astype(jnp.int32)
                   + (lax.shift_right_logical(rng, 24) != 0).astype(jnp.int32))

        scope = jax.named_scope("ph_emit"); scope.__enter__()
        def emit_from(src):
            def ebody(i, _):
                vs = [src[pl.ds((i * _UNROLL + u) * _L, _L)]
                      for u in range(_UNROLL)]
                kd = [v + kmin for v in vs]
                ud = [jnp.bitwise_not(k) ^ _MIN32 for k in kd]
                sg = [lax.shift_right_arithmetic(k, 31) & 0x7FFFFFFF
                      for k in ud]
                bs = [k ^ s for k, s in zip(ud, sg)]
                fs = [plsc.bitcast(b, jnp.float32) for b in bs]
                for u in range(_UNROLL):
                    outv[pl.ds((i * _UNROLL + u) * _L, _L)] = fs[u]
                return 0
            lax.fori_loop(0, _KV // _UNROLL, ebody, 0)

        @pl.when(npasses % 2 == 1)
        def _ea():
            emit_from(bufb)
        @pl.when(npasses % 2 == 0)
        def _eb():
            emit_from(bufa)
        pltpu.sync_copy(outv, out_hbm.at[row])
        scope.__exit__(None, None, None)
        return 0

    pltpu.async_copy(x_hbm.at[row0], xv, sem)
    zero_h2()
    lax.fori_loop(0, _B // 32, row_body, 0)


def _topk_vals(x):
    mesh = plsc.VectorSubcoreMesh(core_axis_name="c", subcore_axis_name="s")
    call = functools.partial(
        pl.kernel,
        out_type=jax.ShapeDtypeStruct((_B, _K), jnp.float32),
        mesh=mesh,
        scratch_types=[
            pltpu.VMEM((_FDIM,), jnp.float32),
            pltpu.VMEM((2048,), jnp.int32),
            pltpu.VMEM((1024 * _L,), jnp.int32),
            pltpu.VMEM((_CAP + 2 * _L,), jnp.int32),
            pltpu.VMEM((_CAP + 2 * _L,), jnp.int32),
            pltpu.VMEM((_K,), jnp.float32),
            pltpu.SMEM((4,), jnp.int32),
            pltpu.SemaphoreType.DMA,
        ],
        compiler_params=pltpu.CompilerParams(needs_layout_passes=False),
    )(_topk_body)
    return call(x)


def kernel(x, cls_token):
    vals = _topk_vals(x)
    rank = jnp.broadcast_to(
        jnp.linspace(0.0, 1.0, _K, dtype=jnp.float32)[None, :], (_B, _K))
    dropout = (vals == 0).astype(jnp.float32)
    tokens = jnp.stack([vals, rank, dropout], axis=-1)
    cls = jnp.broadcast_to(cls_token, (_B, 1, tokens.shape[-1]))
    return jnp.concatenate([cls, tokens], axis=1)
